# baseline (speedup 1.0000x reference)
import functools

import jax
import jax.numpy as jnp
from jax import lax
from jax.experimental import pallas as pl
from jax.experimental.pallas import tpu as pltpu
from jax.experimental.pallas import tpu_sc as plsc

N = 2048
DIM = 768
H = 12
CH = 64
HP = 16
T_PAD = 1 << 20

NC = 2
NS = 16
NW = NC * NS
PER_W = (N * N) // NW
CHUNK = 2048
N_CHUNKS = PER_W // CHUNK
ROWS128 = CHUNK // 128
PSTRIDE = CHUNK + 8

BQ = 128
BK = 512
SCALE = CH ** -0.5
NEG_BIG = -1e30


def _table_body(pt_ref, w_ref, b_ref, o_ref):
    o_ref[...] = (
        lax.dot_general(pt_ref[...], w_ref[...], (((1,), (0,)), ((), ())),
                        preferred_element_type=jnp.float32)
        + b_ref[...]
    )


def _project_table(pre40, w2, b128):
    bt = 1024
    return pl.pallas_call(
        _table_body,
        grid=(T_PAD // 8 // bt,),
        in_specs=[
            pl.BlockSpec((bt, 40), lambda i: (i, 0)),
            pl.BlockSpec((40, 128), lambda i: (0, 0)),
            pl.BlockSpec((1, 128), lambda i: (0, 0)),
        ],
        out_specs=pl.BlockSpec((bt, 128), lambda i: (i, 0)),
        out_shape=jax.ShapeDtypeStruct((T_PAD // 8, 128), jnp.float32),
    )(pre40, w2, b128)


def _qkv_body(x_ref, qw_ref, qb_ref, kvw_ref, kvb_ref, q_ref, kv_ref):
    x = x_ref[...]
    q = lax.dot_general(x, qw_ref[...], (((1,), (1,)), ((), ())),
                        preferred_element_type=jnp.float32) + qb_ref[...]
    q_ref[...] = q * SCALE
    kv_ref[...] = lax.dot_general(x, kvw_ref[...], (((1,), (1,)), ((), ())),
                                  preferred_element_type=jnp.float32) + kvb_ref[...]


def _project_qkv(x, q_w, q_b, kv_w, kv_b):
    bn = 256
    return pl.pallas_call(
        _qkv_body,
        grid=(N // bn,),
        in_specs=[
            pl.BlockSpec((bn, DIM), lambda i: (i, 0)),
            pl.BlockSpec((DIM, DIM), lambda i: (0, 0)),
            pl.BlockSpec((1, DIM), lambda i: (0, 0)),
            pl.BlockSpec((2 * DIM, DIM), lambda i: (0, 0)),
            pl.BlockSpec((1, 2 * DIM), lambda i: (0, 0)),
        ],
        out_specs=[
            pl.BlockSpec((bn, DIM), lambda i: (i, 0)),
            pl.BlockSpec((bn, 2 * DIM), lambda i: (i, 0)),
        ],
        out_shape=[
            jax.ShapeDtypeStruct((N, DIM), jnp.float32),
            jax.ShapeDtypeStruct((N, 2 * DIM), jnp.float32),
        ],
    )(x, q_w, q_b.reshape(1, DIM), kv_w, kv_b.reshape(1, 2 * DIM))


def _sc_gather_body(table_hbm, idx_hbm, out_hbm, idx_v, rows_v, planes_v,
                    sem0, sem1):
    wid = lax.axis_index("s") * NC + lax.axis_index("c")
    row0 = wid * N_CHUNKS
    sems = (sem0, sem1)

    def stage(slot, row):
        pltpu.sync_copy(idx_hbm.at[row], idx_v.at[slot])
        for r in range(ROWS128):
            pltpu.async_copy(
                table_hbm.at[idx_v.at[slot, pl.ds(r * 128, 128)]],
                rows_v.at[slot, pl.ds(r * 128, 128)], sems[slot])

    def drain(slot):
        pltpu.make_async_copy(table_hbm.at[pl.ds(0, CHUNK)],
                              rows_v.at[slot], sems[slot]).wait()

    def process(slot, row):
        strides = lax.iota(jnp.int32, 16) * PSTRIDE

        def g_body(g, carry2):
            p0 = g * 8
            for u in range(8):
                vals = rows_v[slot, p0 + u, :]
                plsc.store_scatter(planes_v, [strides + (p0 + u)], vals)
            return carry2

        lax.fori_loop(0, CHUNK // 8, g_body, 0, unroll=False)
        for h in range(H):
            pltpu.sync_copy(planes_v.at[pl.ds(h * PSTRIDE, CHUNK)],
                            out_hbm.at[h, row])

    stage(0, row0)

    def pair_body(c2, carry):
        base_row = row0 + c2 * 2
        stage(1, base_row + 1)
        drain(0)
        process(0, base_row)

        @pl.when(c2 * 2 + 2 < N_CHUNKS)
        def _prefetch():
            stage(0, base_row + 2)

        drain(1)
        process(1, base_row + 1)
        return carry

    lax.fori_loop(0, N_CHUNKS // 2, pair_body, 0, unroll=False)


@functools.cache
def _get_sc_gather():
    return pl.kernel(
        _sc_gather_body,
        out_type=jax.ShapeDtypeStruct((H, N, N), jnp.float32),
        mesh=plsc.VectorSubcoreMesh(core_axis_name="c", subcore_axis_name="s"),
        scratch_types=[
            pltpu.VMEM((2, CHUNK), jnp.int32),
            pltpu.VMEM((2, CHUNK, HP), jnp.float32),
            pltpu.VMEM((HP * PSTRIDE,), jnp.float32),
            pltpu.SemaphoreType.DMA,
            pltpu.SemaphoreType.DMA,
        ],
        compiler_params=pltpu.CompilerParams(needs_layout_passes=False,
                                             use_tc_tiling_on_sc=False),
    )


def _attn_body(q_ref, kv_ref, bias_ref, bk_ref, bv_ref, pw_ref, pb_ref,
               o_ref, acc_ref, m_ref, l_ref):
    j = pl.program_id(1)

    @pl.when(j == 0)
    def _init():
        m_ref[...] = jnp.full_like(m_ref, NEG_BIG)
        l_ref[...] = jnp.zeros_like(l_ref)
        acc_ref[...] = jnp.zeros_like(acc_ref)

    q = q_ref[...]
    kv = kv_ref[pl.ds(j * BK, BK), :]
    for h in range(H):
        q_h = q[:, h * CH:(h + 1) * CH]
        k_h = kv[:, h * 2 * CH:h * 2 * CH + CH]
        v_h = kv[:, h * 2 * CH + CH:h * 2 * CH + 2 * CH]
        s = lax.dot_general(q_h, k_h, (((1,), (1,)), ((), ())),
                            preferred_element_type=jnp.float32)
        s = s + bias_ref[h]
        m_old = m_ref[:, h:h + 1]
        m_new = jnp.maximum(m_old, jnp.max(s, axis=1, keepdims=True))
        alpha = jnp.exp(m_old - m_new)
        p = jnp.exp(s - m_new)
        l_ref[:, h:h + 1] = (l_ref[:, h:h + 1] * alpha
                             + jnp.sum(p, axis=1, keepdims=True))
        acc_ref[:, h * CH:(h + 1) * CH] = (
            acc_ref[:, h * CH:(h + 1) * CH] * alpha
            + lax.dot_general(p, v_h, (((1,), (0,)), ((), ())),
                              preferred_element_type=jnp.float32))
        m_ref[:, h:h + 1] = m_new

    @pl.when(j == pl.num_programs(1) - 1)
    def _fin():
        cols = []
        for h in range(H):
            q_h = q[:, h * CH:(h + 1) * CH]
            bl = jnp.sum(q_h * bk_ref[:, h * CH:(h + 1) * CH],
                         axis=1, keepdims=True)
            m_old = m_ref[:, h:h + 1]
            m_fin = jnp.maximum(m_old, bl)
            a0 = jnp.exp(m_old - m_fin)
            ab = jnp.exp(bl - m_fin)
            l_fin = l_ref[:, h:h + 1] * a0 + ab
            acc_h = (acc_ref[:, h * CH:(h + 1) * CH] * a0
                     + ab * bv_ref[:, h * CH:(h + 1) * CH])
            cols.append(acc_h / l_fin)
        obuf = jnp.concatenate(cols, axis=1)
        o_ref[...] = (
            lax.dot_general(obuf, pw_ref[...], (((1,), (1,)), ((), ())),
                            preferred_element_type=jnp.float32)
            + pb_ref[...]
        )


def _attention(q2d, kv2d, bias3, blank_k, blank_v, proj_w, proj_b):
    return pl.pallas_call(
        _attn_body,
        grid=(N // BQ, N // BK),
        in_specs=[
            pl.BlockSpec((BQ, DIM), lambda i, j: (i, 0)),
            pl.BlockSpec((N, 2 * DIM), lambda i, j: (0, 0)),
            pl.BlockSpec((H, BQ, BK), lambda i, j: (0, i, j)),
            pl.BlockSpec((1, DIM), lambda i, j: (0, 0)),
            pl.BlockSpec((1, DIM), lambda i, j: (0, 0)),
            pl.BlockSpec((DIM, DIM), lambda i, j: (0, 0)),
            pl.BlockSpec((1, DIM), lambda i, j: (0, 0)),
        ],
        out_specs=pl.BlockSpec((BQ, DIM), lambda i, j: (i, 0)),
        out_shape=jax.ShapeDtypeStruct((N, DIM), jnp.float32),
        scratch_shapes=[
            pltpu.VMEM((BQ, DIM), jnp.float32),
            pltpu.VMEM((BQ, HP), jnp.float32),
            pltpu.VMEM((BQ, HP), jnp.float32),
        ],
        compiler_params=pltpu.CompilerParams(
            dimension_semantics=("arbitrary", "arbitrary")),
    )(q2d, kv2d, bias3, blank_k, blank_v, proj_w, proj_b)


def kernel(feat, member_idx, cluster_mask, pe_idx, global_attn, q_w, q_b,
           kv_w, kv_b, blank_k, blank_v, pos_w, pos_b, proj_w, proj_b,
           pre_table):
    x = feat[0]
    idxnn = pe_idx.reshape(N, N).astype(jnp.int32)

    t = pre_table.shape[0]
    flat = jnp.pad(pre_table.reshape(-1), (0, (T_PAD - t) * 5))
    pre40 = flat.reshape(T_PAD // 8, 40)
    wp = jnp.pad(pos_w, ((0, HP - H), (0, 0)))
    w2 = jnp.zeros((40, 128), jnp.float32)
    for jj in range(8):
        w2 = w2.at[jj * 5:(jj + 1) * 5, jj * HP:(jj + 1) * HP].set(wp.T)
    b128 = jnp.tile(jnp.pad(pos_b, (0, HP - H)), 8).reshape(1, 128)

    table_pk = _project_table(pre40, w2, b128)
    table16 = table_pk.reshape(T_PAD, HP)
    bias3 = _get_sc_gather()(table16, idxnn)

    q2d, kv2d = _project_qkv(x, q_w, q_b, kv_w, kv_b)

    out = _attention(q2d, kv2d, bias3,
                     blank_k.reshape(1, DIM), blank_v.reshape(1, DIM),
                     proj_w, proj_b.reshape(1, DIM))
    return out.reshape(1, N, DIM)


# speedup vs baseline: 1.0405x; 1.0405x over previous
import functools

import jax
import jax.numpy as jnp
from jax import lax
from jax.experimental import pallas as pl
from jax.experimental.pallas import tpu as pltpu
from jax.experimental.pallas import tpu_sc as plsc

N = 2048
DIM = 768
H = 12
CH = 64
HP = 16
T_PAD = 1 << 20

NC = 2
NS = 16
NW = NC * NS
PER_W = (N * N) // NW
CHUNK = 2048
N_CHUNKS = PER_W // CHUNK
ROWS128 = CHUNK // 128
PSTRIDE = CHUNK + 8

BQ = 128
BK = 512
SCALE = CH ** -0.5
NEG_BIG = -1e30


def _table_body(pt_ref, w_ref, b_ref, o_ref):
    o_ref[...] = (
        lax.dot_general(pt_ref[...], w_ref[...], (((1,), (0,)), ((), ())),
                        preferred_element_type=jnp.float32)
        + b_ref[...]
    )


def _project_table(pre40, w2, b128):
    bt = 1024
    return pl.pallas_call(
        _table_body,
        grid=(T_PAD // 8 // bt,),
        in_specs=[
            pl.BlockSpec((bt, 40), lambda i: (i, 0)),
            pl.BlockSpec((40, 128), lambda i: (0, 0)),
            pl.BlockSpec((1, 128), lambda i: (0, 0)),
        ],
        out_specs=pl.BlockSpec((bt, 128), lambda i: (i, 0)),
        out_shape=jax.ShapeDtypeStruct((T_PAD // 8, 128), jnp.float32),
    )(pre40, w2, b128)


def _qkv_body(x_ref, qw_ref, qb_ref, kvw_ref, kvb_ref, q_ref, kv_ref):
    x = x_ref[...]
    q = lax.dot_general(x, qw_ref[...], (((1,), (1,)), ((), ())),
                        preferred_element_type=jnp.float32) + qb_ref[...]
    q_ref[...] = q * SCALE
    kv_ref[...] = lax.dot_general(x, kvw_ref[...], (((1,), (1,)), ((), ())),
                                  preferred_element_type=jnp.float32) + kvb_ref[...]


def _project_qkv(x, q_w, q_b, kv_w, kv_b):
    bn = 256
    return pl.pallas_call(
        _qkv_body,
        grid=(N // bn,),
        in_specs=[
            pl.BlockSpec((bn, DIM), lambda i: (i, 0)),
            pl.BlockSpec((DIM, DIM), lambda i: (0, 0)),
            pl.BlockSpec((1, DIM), lambda i: (0, 0)),
            pl.BlockSpec((2 * DIM, DIM), lambda i: (0, 0)),
            pl.BlockSpec((1, 2 * DIM), lambda i: (0, 0)),
        ],
        out_specs=[
            pl.BlockSpec((bn, DIM), lambda i: (i, 0)),
            pl.BlockSpec((bn, 2 * DIM), lambda i: (i, 0)),
        ],
        out_shape=[
            jax.ShapeDtypeStruct((N, DIM), jnp.float32),
            jax.ShapeDtypeStruct((N, 2 * DIM), jnp.float32),
        ],
    )(x, q_w, q_b.reshape(1, DIM), kv_w, kv_b.reshape(1, 2 * DIM))


def _sc_gather_body(half, table_hbm, idx_hbm, out_hbm, idx_v, rows_v,
                    planes_v, sem0, sem1):
    wid = lax.axis_index("s") * NC + lax.axis_index("c")
    nch = N_CHUNKS // 2
    row0 = half * (N // 2) + wid * nch
    sems = (sem0, sem1)

    def stage(slot, row):
        pltpu.sync_copy(idx_hbm.at[row], idx_v.at[slot])
        for r in range(ROWS128):
            pltpu.async_copy(
                table_hbm.at[idx_v.at[slot, pl.ds(r * 128, 128)]],
                rows_v.at[slot, pl.ds(r * 128, 128)], sems[slot])

    def drain(slot):
        pltpu.make_async_copy(table_hbm.at[pl.ds(0, CHUNK)],
                              rows_v.at[slot], sems[slot]).wait()

    def process(slot, row):
        strides = lax.iota(jnp.int32, 16) * PSTRIDE

        def g_body(g, carry2):
            p0 = g * 8
            for u in range(8):
                vals = rows_v[slot, p0 + u, :]
                plsc.store_scatter(planes_v, [strides + (p0 + u)], vals)
            return carry2

        lax.fori_loop(0, CHUNK // 8, g_body, 0, unroll=False)
        for h in range(H):
            pltpu.sync_copy(planes_v.at[pl.ds(h * PSTRIDE, CHUNK)],
                            out_hbm.at[h, row - half * (N // 2)])

    stage(0, row0)

    def pair_body(c2, carry):
        base_row = row0 + c2 * 2
        stage(1, base_row + 1)
        drain(0)
        process(0, base_row)

        @pl.when(c2 * 2 + 2 < nch)
        def _prefetch():
            stage(0, base_row + 2)

        drain(1)
        process(1, base_row + 1)
        return carry

    lax.fori_loop(0, nch // 2, pair_body, 0, unroll=False)


@functools.cache
def _get_sc_gather(half):
    return pl.kernel(
        functools.partial(_sc_gather_body, half),
        out_type=jax.ShapeDtypeStruct((H, N // 2, N), jnp.float32),
        mesh=plsc.VectorSubcoreMesh(core_axis_name="c", subcore_axis_name="s"),
        scratch_types=[
            pltpu.VMEM((2, CHUNK), jnp.int32),
            pltpu.VMEM((2, CHUNK, HP), jnp.float32),
            pltpu.VMEM((HP * PSTRIDE,), jnp.float32),
            pltpu.SemaphoreType.DMA,
            pltpu.SemaphoreType.DMA,
        ],
        compiler_params=pltpu.CompilerParams(needs_layout_passes=False,
                                             use_tc_tiling_on_sc=False),
    )


def _attn_body(q_ref, kv_ref, bias_ref, bk_ref, bv_ref, pw_ref, pb_ref,
               o_ref, acc_ref, m_ref, l_ref):
    j = pl.program_id(1)

    @pl.when(j == 0)
    def _init():
        m_ref[...] = jnp.full_like(m_ref, NEG_BIG)
        l_ref[...] = jnp.zeros_like(l_ref)
        acc_ref[...] = jnp.zeros_like(acc_ref)

    q = q_ref[...]
    kv = kv_ref[pl.ds(j * BK, BK), :]
    for h in range(H):
        q_h = q[:, h * CH:(h + 1) * CH]
        k_h = kv[:, h * 2 * CH:h * 2 * CH + CH]
        v_h = kv[:, h * 2 * CH + CH:h * 2 * CH + 2 * CH]
        s = lax.dot_general(q_h, k_h, (((1,), (1,)), ((), ())),
                            preferred_element_type=jnp.float32)
        s = s + bias_ref[h]
        m_old = m_ref[:, h:h + 1]
        m_new = jnp.maximum(m_old, jnp.max(s, axis=1, keepdims=True))
        alpha = jnp.exp(m_old - m_new)
        p = jnp.exp(s - m_new)
        l_ref[:, h:h + 1] = (l_ref[:, h:h + 1] * alpha
                             + jnp.sum(p, axis=1, keepdims=True))
        acc_ref[:, h * CH:(h + 1) * CH] = (
            acc_ref[:, h * CH:(h + 1) * CH] * alpha
            + lax.dot_general(p, v_h, (((1,), (0,)), ((), ())),
                              preferred_element_type=jnp.float32))
        m_ref[:, h:h + 1] = m_new

    @pl.when(j == pl.num_programs(1) - 1)
    def _fin():
        cols = []
        for h in range(H):
            q_h = q[:, h * CH:(h + 1) * CH]
            bl = jnp.sum(q_h * bk_ref[:, h * CH:(h + 1) * CH],
                         axis=1, keepdims=True)
            m_old = m_ref[:, h:h + 1]
            m_fin = jnp.maximum(m_old, bl)
            a0 = jnp.exp(m_old - m_fin)
            ab = jnp.exp(bl - m_fin)
            l_fin = l_ref[:, h:h + 1] * a0 + ab
            acc_h = (acc_ref[:, h * CH:(h + 1) * CH] * a0
                     + ab * bv_ref[:, h * CH:(h + 1) * CH])
            cols.append(acc_h / l_fin)
        obuf = jnp.concatenate(cols, axis=1)
        o_ref[...] = (
            lax.dot_general(obuf, pw_ref[...], (((1,), (1,)), ((), ())),
                            preferred_element_type=jnp.float32)
            + pb_ref[...]
        )


def _attention(half, q2d, kv2d, bias3, blank_k, blank_v, proj_w, proj_b):
    ioff = half * (N // 2) // BQ
    return pl.pallas_call(
        _attn_body,
        grid=(N // 2 // BQ, N // BK),
        in_specs=[
            pl.BlockSpec((BQ, DIM), lambda i, j: (i + ioff, 0)),
            pl.BlockSpec((N, 2 * DIM), lambda i, j: (0, 0)),
            pl.BlockSpec((H, BQ, BK), lambda i, j: (0, i, j)),
            pl.BlockSpec((1, DIM), lambda i, j: (0, 0)),
            pl.BlockSpec((1, DIM), lambda i, j: (0, 0)),
            pl.BlockSpec((DIM, DIM), lambda i, j: (0, 0)),
            pl.BlockSpec((1, DIM), lambda i, j: (0, 0)),
        ],
        out_specs=pl.BlockSpec((BQ, DIM), lambda i, j: (i, 0)),
        out_shape=jax.ShapeDtypeStruct((N // 2, DIM), jnp.float32),
        scratch_shapes=[
            pltpu.VMEM((BQ, DIM), jnp.float32),
            pltpu.VMEM((BQ, HP), jnp.float32),
            pltpu.VMEM((BQ, HP), jnp.float32),
        ],
        compiler_params=pltpu.CompilerParams(
            dimension_semantics=("arbitrary", "arbitrary")),
    )(q2d, kv2d, bias3, blank_k, blank_v, proj_w, proj_b)


def kernel(feat, member_idx, cluster_mask, pe_idx, global_attn, q_w, q_b,
           kv_w, kv_b, blank_k, blank_v, pos_w, pos_b, proj_w, proj_b,
           pre_table):
    x = feat[0]
    idxnn = pe_idx.reshape(N, N).astype(jnp.int32)

    t = pre_table.shape[0]
    flat = jnp.pad(pre_table.reshape(-1), (0, (T_PAD - t) * 5))
    pre40 = flat.reshape(T_PAD // 8, 40)
    wp = jnp.pad(pos_w, ((0, HP - H), (0, 0)))
    w2 = jnp.zeros((40, 128), jnp.float32)
    for jj in range(8):
        w2 = w2.at[jj * 5:(jj + 1) * 5, jj * HP:(jj + 1) * HP].set(wp.T)
    b128 = jnp.tile(jnp.pad(pos_b, (0, HP - H)), 8).reshape(1, 128)

    table_pk = _project_table(pre40, w2, b128)
    table16 = table_pk.reshape(T_PAD, HP)
    q2d, kv2d = _project_qkv(x, q_w, q_b, kv_w, kv_b)

    bk1 = blank_k.reshape(1, DIM)
    bv1 = blank_v.reshape(1, DIM)
    pb1 = proj_b.reshape(1, DIM)
    bias_a = _get_sc_gather(0)(table16, idxnn)
    bias_b = _get_sc_gather(1)(table16, idxnn)
    out_a = _attention(0, q2d, kv2d, bias_a, bk1, bv1, proj_w, pb1)
    out_b = _attention(1, q2d, kv2d, bias_b, bk1, bv1, proj_w, pb1)
    out = jnp.concatenate([out_a, out_b], axis=0)
    return out.reshape(1, N, DIM)


# speedup vs baseline: 1.1261x; 1.0823x over previous
import functools

import jax
import jax.numpy as jnp
from jax import lax
from jax.experimental import pallas as pl
from jax.experimental.pallas import tpu as pltpu
from jax.experimental.pallas import tpu_sc as plsc

N = 2048
DIM = 768
H = 12
CH = 64
HP = 16
T_PAD = 1 << 20

NC = 2
NS = 16
NW = NC * NS
PER_W = (N * N) // NW
CHUNK = 2048
N_CHUNKS = PER_W // CHUNK
ROWS128 = CHUNK // 128
PSTRIDE = CHUNK + 8

BQ = 128
BK = 512
SCALE = CH ** -0.5
NEG_BIG = -1e30


def _table_body(pt_ref, w_ref, b_ref, o_ref):
    o_ref[...] = (
        lax.dot_general(pt_ref[...], w_ref[...], (((1,), (0,)), ((), ())),
                        preferred_element_type=jnp.float32)
        + b_ref[...]
    )


def _project_table(pre40, w2, b128):
    bt = 1024
    return pl.pallas_call(
        _table_body,
        grid=(T_PAD // 8 // bt,),
        in_specs=[
            pl.BlockSpec((bt, 40), lambda i: (i, 0)),
            pl.BlockSpec((40, 128), lambda i: (0, 0)),
            pl.BlockSpec((1, 128), lambda i: (0, 0)),
        ],
        out_specs=pl.BlockSpec((bt, 128), lambda i: (i, 0)),
        out_shape=jax.ShapeDtypeStruct((T_PAD // 8, 128), jnp.float32),
    )(pre40, w2, b128)


def _qkv_body(x_ref, qw_ref, qb_ref, kvw_ref, kvb_ref, q_ref, kv_ref):
    x = x_ref[...]
    q = lax.dot_general(x, qw_ref[...], (((1,), (1,)), ((), ())),
                        preferred_element_type=jnp.float32) + qb_ref[...]
    q_ref[...] = q * SCALE
    kv_ref[...] = lax.dot_general(x, kvw_ref[...], (((1,), (1,)), ((), ())),
                                  preferred_element_type=jnp.float32) + kvb_ref[...]


def _project_qkv(x, q_w, q_b, kv_w, kv_b):
    bn = 256
    return pl.pallas_call(
        _qkv_body,
        grid=(N // bn,),
        in_specs=[
            pl.BlockSpec((bn, DIM), lambda i: (i, 0)),
            pl.BlockSpec((DIM, DIM), lambda i: (0, 0)),
            pl.BlockSpec((1, DIM), lambda i: (0, 0)),
            pl.BlockSpec((2 * DIM, DIM), lambda i: (0, 0)),
            pl.BlockSpec((1, 2 * DIM), lambda i: (0, 0)),
        ],
        out_specs=[
            pl.BlockSpec((bn, DIM), lambda i: (i, 0)),
            pl.BlockSpec((bn, 2 * DIM), lambda i: (i, 0)),
        ],
        out_shape=[
            jax.ShapeDtypeStruct((N, DIM), jnp.float32),
            jax.ShapeDtypeStruct((N, 2 * DIM), jnp.float32),
        ],
    )(x, q_w, q_b.reshape(1, DIM), kv_w, kv_b.reshape(1, 2 * DIM))


NSPLIT = 4


def _sc_gather_body(part, table_hbm, idx_hbm, out_hbm, idx_v, rows_v,
                    planes_v, sem0, sem1):
    wid = lax.axis_index("s") * NC + lax.axis_index("c")
    nch = N_CHUNKS // NSPLIT
    row0 = part * (N // NSPLIT) + wid * nch
    sems = (sem0, sem1)

    def stage(slot, row):
        pltpu.sync_copy(idx_hbm.at[row], idx_v.at[slot])
        for r in range(ROWS128):
            pltpu.async_copy(
                table_hbm.at[idx_v.at[slot, pl.ds(r * 128, 128)]],
                rows_v.at[slot, pl.ds(r * 128, 128)], sems[slot])

    def drain(slot):
        pltpu.make_async_copy(table_hbm.at[pl.ds(0, CHUNK)],
                              rows_v.at[slot], sems[slot]).wait()

    def process(slot, row):
        strides = lax.iota(jnp.int32, 16) * PSTRIDE

        def g_body(g, carry2):
            p0 = g * 8
            for u in range(8):
                vals = rows_v[slot, p0 + u, :]
                plsc.store_scatter(planes_v, [strides + (p0 + u)], vals)
            return carry2

        lax.fori_loop(0, CHUNK // 8, g_body, 0, unroll=False)
        for h in range(H):
            pltpu.sync_copy(planes_v.at[pl.ds(h * PSTRIDE, CHUNK)],
                            out_hbm.at[h, row - part * (N // NSPLIT)])

    stage(0, row0)

    def pair_body(c2, carry):
        base_row = row0 + c2 * 2
        stage(1, base_row + 1)
        drain(0)
        process(0, base_row)

        @pl.when(c2 * 2 + 2 < nch)
        def _prefetch():
            stage(0, base_row + 2)

        drain(1)
        process(1, base_row + 1)
        return carry

    lax.fori_loop(0, nch // 2, pair_body, 0, unroll=False)


@functools.cache
def _get_sc_gather(part):
    return pl.kernel(
        functools.partial(_sc_gather_body, part),
        out_type=jax.ShapeDtypeStruct((H, N // NSPLIT, N), jnp.float32),
        mesh=plsc.VectorSubcoreMesh(core_axis_name="c", subcore_axis_name="s"),
        scratch_types=[
            pltpu.VMEM((2, CHUNK), jnp.int32),
            pltpu.VMEM((2, CHUNK, HP), jnp.float32),
            pltpu.VMEM((HP * PSTRIDE,), jnp.float32),
            pltpu.SemaphoreType.DMA,
            pltpu.SemaphoreType.DMA,
        ],
        compiler_params=pltpu.CompilerParams(needs_layout_passes=False,
                                             use_tc_tiling_on_sc=False),
    )


def _attn_body(q_ref, kv_ref, bias_ref, bk_ref, bv_ref, pw_ref, pb_ref,
               o_ref, acc_ref, m_ref, l_ref):
    j = pl.program_id(1)

    @pl.when(j == 0)
    def _init():
        m_ref[...] = jnp.full_like(m_ref, NEG_BIG)
        l_ref[...] = jnp.zeros_like(l_ref)
        acc_ref[...] = jnp.zeros_like(acc_ref)

    q = q_ref[...]
    kv = kv_ref[pl.ds(j * BK, BK), :]
    for h in range(H):
        q_h = q[:, h * CH:(h + 1) * CH]
        k_h = kv[:, h * 2 * CH:h * 2 * CH + CH]
        v_h = kv[:, h * 2 * CH + CH:h * 2 * CH + 2 * CH]
        s = lax.dot_general(q_h, k_h, (((1,), (1,)), ((), ())),
                            preferred_element_type=jnp.float32)
        s = s + bias_ref[h]
        m_old = m_ref[:, h:h + 1]
        m_new = jnp.maximum(m_old, jnp.max(s, axis=1, keepdims=True))
        alpha = jnp.exp(m_old - m_new)
        p = jnp.exp(s - m_new)
        l_ref[:, h:h + 1] = (l_ref[:, h:h + 1] * alpha
                             + jnp.sum(p, axis=1, keepdims=True))
        acc_ref[:, h * CH:(h + 1) * CH] = (
            acc_ref[:, h * CH:(h + 1) * CH] * alpha
            + lax.dot_general(p, v_h, (((1,), (0,)), ((), ())),
                              preferred_element_type=jnp.float32))
        m_ref[:, h:h + 1] = m_new

    @pl.when(j == pl.num_programs(1) - 1)
    def _fin():
        cols = []
        for h in range(H):
            q_h = q[:, h * CH:(h + 1) * CH]
            bl = jnp.sum(q_h * bk_ref[:, h * CH:(h + 1) * CH],
                         axis=1, keepdims=True)
            m_old = m_ref[:, h:h + 1]
            m_fin = jnp.maximum(m_old, bl)
            a0 = jnp.exp(m_old - m_fin)
            ab = jnp.exp(bl - m_fin)
            l_fin = l_ref[:, h:h + 1] * a0 + ab
            acc_h = (acc_ref[:, h * CH:(h + 1) * CH] * a0
                     + ab * bv_ref[:, h * CH:(h + 1) * CH])
            cols.append(acc_h / l_fin)
        obuf = jnp.concatenate(cols, axis=1)
        o_ref[...] = (
            lax.dot_general(obuf, pw_ref[...], (((1,), (1,)), ((), ())),
                            preferred_element_type=jnp.float32)
            + pb_ref[...]
        )


def _attention(part, q2d, kv2d, bias3, blank_k, blank_v, proj_w, proj_b):
    ioff = part * (N // NSPLIT) // BQ
    return pl.pallas_call(
        _attn_body,
        grid=(N // NSPLIT // BQ, N // BK),
        in_specs=[
            pl.BlockSpec((BQ, DIM), lambda i, j: (i + ioff, 0)),
            pl.BlockSpec((N, 2 * DIM), lambda i, j: (0, 0)),
            pl.BlockSpec((H, BQ, BK), lambda i, j: (0, i, j)),
            pl.BlockSpec((1, DIM), lambda i, j: (0, 0)),
            pl.BlockSpec((1, DIM), lambda i, j: (0, 0)),
            pl.BlockSpec((DIM, DIM), lambda i, j: (0, 0)),
            pl.BlockSpec((1, DIM), lambda i, j: (0, 0)),
        ],
        out_specs=pl.BlockSpec((BQ, DIM), lambda i, j: (i, 0)),
        out_shape=jax.ShapeDtypeStruct((N // NSPLIT, DIM), jnp.float32),
        scratch_shapes=[
            pltpu.VMEM((BQ, DIM), jnp.float32),
            pltpu.VMEM((BQ, HP), jnp.float32),
            pltpu.VMEM((BQ, HP), jnp.float32),
        ],
        compiler_params=pltpu.CompilerParams(
            dimension_semantics=("arbitrary", "arbitrary")),
    )(q2d, kv2d, bias3, blank_k, blank_v, proj_w, proj_b)


def kernel(feat, member_idx, cluster_mask, pe_idx, global_attn, q_w, q_b,
           kv_w, kv_b, blank_k, blank_v, pos_w, pos_b, proj_w, proj_b,
           pre_table):
    x = feat[0]
    idxnn = pe_idx.reshape(N, N).astype(jnp.int32)

    t = pre_table.shape[0]
    flat = jnp.pad(pre_table.reshape(-1), (0, (T_PAD - t) * 5))
    pre40 = flat.reshape(T_PAD // 8, 40)
    wp = jnp.pad(pos_w, ((0, HP - H), (0, 0)))
    w2 = jnp.zeros((40, 128), jnp.float32)
    for jj in range(8):
        w2 = w2.at[jj * 5:(jj + 1) * 5, jj * HP:(jj + 1) * HP].set(wp.T)
    b128 = jnp.tile(jnp.pad(pos_b, (0, HP - H)), 8).reshape(1, 128)

    table_pk = _project_table(pre40, w2, b128)
    table16 = table_pk.reshape(T_PAD, HP)
    q2d, kv2d = _project_qkv(x, q_w, q_b, kv_w, kv_b)

    bk1 = blank_k.reshape(1, DIM)
    bv1 = blank_v.reshape(1, DIM)
    pb1 = proj_b.reshape(1, DIM)
    outs = []
    for q in range(NSPLIT):
        idx_q = idxnn
        if q >= 2:
            idx_q, _ = lax.optimization_barrier((idxnn, outs[q - 2]))
        bias_q = _get_sc_gather(q)(table16, idx_q)
        outs.append(_attention(q, q2d, kv2d, bias_q, bk1, bv1, proj_w, pb1))
    out = jnp.concatenate(outs, axis=0)
    return out.reshape(1, N, DIM)


# speedup vs baseline: 1.2467x; 1.1071x over previous
import functools

import jax
import jax.numpy as jnp
from jax import lax
from jax.experimental import pallas as pl
from jax.experimental.pallas import tpu as pltpu
from jax.experimental.pallas import tpu_sc as plsc

N = 2048
DIM = 768
H = 12
CH = 64
HP = 16
T_PAD = 1 << 20

NC = 2
NS = 16
NW = NC * NS
PER_W = (N * N) // NW
CHUNK = 2048
N_CHUNKS = PER_W // CHUNK
ROWS128 = CHUNK // 128
PSTRIDE = CHUNK + 8

BQ = 128
BK = 512
SCALE = CH ** -0.5
NEG_BIG = -1e30


def _table_body(pt_ref, w_ref, b_ref, o_ref):
    o_ref[...] = (
        lax.dot_general(pt_ref[...], w_ref[...], (((1,), (0,)), ((), ())),
                        preferred_element_type=jnp.float32)
        + b_ref[...]
    )


def _project_table(pre40, w2, b128):
    bt = 1024
    return pl.pallas_call(
        _table_body,
        grid=(T_PAD // 8 // bt,),
        in_specs=[
            pl.BlockSpec((bt, 40), lambda i: (i, 0)),
            pl.BlockSpec((40, 128), lambda i: (0, 0)),
            pl.BlockSpec((1, 128), lambda i: (0, 0)),
        ],
        out_specs=pl.BlockSpec((bt, 128), lambda i: (i, 0)),
        out_shape=jax.ShapeDtypeStruct((T_PAD // 8, 128), jnp.float32),
    )(pre40, w2, b128)


def _qkv_body(x_ref, qw_ref, qb_ref, kvw_ref, kvb_ref, q_ref, kv_ref):
    x = x_ref[...]
    q = lax.dot_general(x, qw_ref[...], (((1,), (1,)), ((), ())),
                        preferred_element_type=jnp.float32) + qb_ref[...]
    q_ref[...] = q * SCALE
    kv_ref[...] = lax.dot_general(x, kvw_ref[...], (((1,), (1,)), ((), ())),
                                  preferred_element_type=jnp.float32) + kvb_ref[...]


def _project_qkv(x, q_w, q_b, kv_w, kv_b):
    bn = 256
    return pl.pallas_call(
        _qkv_body,
        grid=(N // bn,),
        in_specs=[
            pl.BlockSpec((bn, DIM), lambda i: (i, 0)),
            pl.BlockSpec((DIM, DIM), lambda i: (0, 0)),
            pl.BlockSpec((1, DIM), lambda i: (0, 0)),
            pl.BlockSpec((2 * DIM, DIM), lambda i: (0, 0)),
            pl.BlockSpec((1, 2 * DIM), lambda i: (0, 0)),
        ],
        out_specs=[
            pl.BlockSpec((bn, DIM), lambda i: (i, 0)),
            pl.BlockSpec((bn, 2 * DIM), lambda i: (i, 0)),
        ],
        out_shape=[
            jax.ShapeDtypeStruct((N, DIM), jnp.float32),
            jax.ShapeDtypeStruct((N, 2 * DIM), jnp.float32),
        ],
    )(x, q_w, q_b.reshape(1, DIM), kv_w, kv_b.reshape(1, 2 * DIM))


NSPLIT = 4


def _sc_gather_body(part, table_hbm, idx_hbm, tok_hbm, out_hbm, idx_v, rows_v,
                    planes_v, sem0, sem1):
    del tok_hbm
    wid = lax.axis_index("s") * NC + lax.axis_index("c")
    nch = N_CHUNKS // NSPLIT
    row0 = part * (N // NSPLIT) + wid * nch
    sems = (sem0, sem1)

    def stage(slot, row):
        pltpu.sync_copy(idx_hbm.at[row], idx_v.at[slot])
        for r in range(ROWS128):
            pltpu.async_copy(
                table_hbm.at[idx_v.at[slot, pl.ds(r * 128, 128)]],
                rows_v.at[slot, pl.ds(r * 128, 128)], sems[slot])

    def drain(slot):
        pltpu.make_async_copy(table_hbm.at[pl.ds(0, CHUNK)],
                              rows_v.at[slot], sems[slot]).wait()

    def process(slot, row):
        strides = lax.iota(jnp.int32, 16) * PSTRIDE

        def g_body(g, carry2):
            p0 = g * 8
            for u in range(8):
                vals = rows_v[slot, p0 + u, :]
                plsc.store_scatter(planes_v, [strides + (p0 + u)], vals)
            return carry2

        lax.fori_loop(0, CHUNK // 8, g_body, 0, unroll=False)
        for h in range(H):
            pltpu.sync_copy(planes_v.at[pl.ds(h * PSTRIDE, CHUNK)],
                            out_hbm.at[h, row - part * (N // NSPLIT)])

    stage(0, row0)

    def pair_body(c2, carry):
        base_row = row0 + c2 * 2
        stage(1, base_row + 1)
        drain(0)
        process(0, base_row)

        @pl.when(c2 * 2 + 2 < nch)
        def _prefetch():
            stage(0, base_row + 2)

        drain(1)
        process(1, base_row + 1)
        return carry

    lax.fori_loop(0, nch // 2, pair_body, 0, unroll=False)


@functools.cache
def _get_sc_gather(part):
    return pl.kernel(
        functools.partial(_sc_gather_body, part),
        out_type=jax.ShapeDtypeStruct((H, N // NSPLIT, N), jnp.float32),
        mesh=plsc.VectorSubcoreMesh(core_axis_name="c", subcore_axis_name="s"),
        scratch_types=[
            pltpu.VMEM((2, CHUNK), jnp.int32),
            pltpu.VMEM((2, CHUNK, HP), jnp.float32),
            pltpu.VMEM((HP * PSTRIDE,), jnp.float32),
            pltpu.SemaphoreType.DMA,
            pltpu.SemaphoreType.DMA,
        ],
        compiler_params=pltpu.CompilerParams(needs_layout_passes=False,
                                             use_tc_tiling_on_sc=False),
    )


def _attn_body(q_ref, kv_ref, bias_ref, bk_ref, bv_ref, pw_ref, pb_ref,
               o_ref, acc_ref, m_ref, l_ref):
    j = pl.program_id(1)

    @pl.when(j == 0)
    def _init():
        m_ref[...] = jnp.full_like(m_ref, NEG_BIG)
        l_ref[...] = jnp.zeros_like(l_ref)
        acc_ref[...] = jnp.zeros_like(acc_ref)

    q = q_ref[...]
    kv = kv_ref[pl.ds(j * BK, BK), :]
    for h in range(H):
        q_h = q[:, h * CH:(h + 1) * CH]
        k_h = kv[:, h * 2 * CH:h * 2 * CH + CH]
        v_h = kv[:, h * 2 * CH + CH:h * 2 * CH + 2 * CH]
        s = lax.dot_general(q_h, k_h, (((1,), (1,)), ((), ())),
                            preferred_element_type=jnp.float32)
        s = s + bias_ref[h]
        m_old = m_ref[:, h:h + 1]
        m_new = jnp.maximum(m_old, jnp.max(s, axis=1, keepdims=True))
        alpha = jnp.exp(m_old - m_new)
        p = jnp.exp(s - m_new)
        l_ref[:, h:h + 1] = (l_ref[:, h:h + 1] * alpha
                             + jnp.sum(p, axis=1, keepdims=True))
        acc_ref[:, h * CH:(h + 1) * CH] = (
            acc_ref[:, h * CH:(h + 1) * CH] * alpha
            + lax.dot_general(p, v_h, (((1,), (0,)), ((), ())),
                              preferred_element_type=jnp.float32))
        m_ref[:, h:h + 1] = m_new

    @pl.when(j == pl.num_programs(1) - 1)
    def _fin():
        cols = []
        for h in range(H):
            q_h = q[:, h * CH:(h + 1) * CH]
            bl = jnp.sum(q_h * bk_ref[:, h * CH:(h + 1) * CH],
                         axis=1, keepdims=True)
            m_old = m_ref[:, h:h + 1]
            m_fin = jnp.maximum(m_old, bl)
            a0 = jnp.exp(m_old - m_fin)
            ab = jnp.exp(bl - m_fin)
            l_fin = l_ref[:, h:h + 1] * a0 + ab
            acc_h = (acc_ref[:, h * CH:(h + 1) * CH] * a0
                     + ab * bv_ref[:, h * CH:(h + 1) * CH])
            cols.append(acc_h / l_fin)
        obuf = jnp.concatenate(cols, axis=1)
        o_ref[...] = (
            lax.dot_general(obuf, pw_ref[...], (((1,), (1,)), ((), ())),
                            preferred_element_type=jnp.float32)
            + pb_ref[...]
        )


def _attention(part, q2d, kv2d, bias3, blank_k, blank_v, proj_w, proj_b):
    ioff = part * (N // NSPLIT) // BQ
    return pl.pallas_call(
        _attn_body,
        grid=(N // NSPLIT // BQ, N // BK),
        in_specs=[
            pl.BlockSpec((BQ, DIM), lambda i, j: (i + ioff, 0)),
            pl.BlockSpec((N, 2 * DIM), lambda i, j: (0, 0)),
            pl.BlockSpec((H, BQ, BK), lambda i, j: (0, i, j)),
            pl.BlockSpec((1, DIM), lambda i, j: (0, 0)),
            pl.BlockSpec((1, DIM), lambda i, j: (0, 0)),
            pl.BlockSpec((DIM, DIM), lambda i, j: (0, 0)),
            pl.BlockSpec((1, DIM), lambda i, j: (0, 0)),
        ],
        out_specs=pl.BlockSpec((BQ, DIM), lambda i, j: (i, 0)),
        out_shape=jax.ShapeDtypeStruct((N // NSPLIT, DIM), jnp.float32),
        scratch_shapes=[
            pltpu.VMEM((BQ, DIM), jnp.float32),
            pltpu.VMEM((BQ, HP), jnp.float32),
            pltpu.VMEM((BQ, HP), jnp.float32),
        ],
        compiler_params=pltpu.CompilerParams(
            dimension_semantics=("arbitrary", "arbitrary")),
    )(q2d, kv2d, bias3, blank_k, blank_v, proj_w, proj_b)


def kernel(feat, member_idx, cluster_mask, pe_idx, global_attn, q_w, q_b,
           kv_w, kv_b, blank_k, blank_v, pos_w, pos_b, proj_w, proj_b,
           pre_table):
    x = feat[0]
    idxnn = pe_idx.reshape(N, N).astype(jnp.int32)

    t = pre_table.shape[0]
    flat = jnp.pad(pre_table.reshape(-1), (0, (T_PAD - t) * 5))
    pre40 = flat.reshape(T_PAD // 8, 40)
    wp = jnp.pad(pos_w, ((0, HP - H), (0, 0)))
    w2 = jnp.zeros((40, 128), jnp.float32)
    for jj in range(8):
        w2 = w2.at[jj * 5:(jj + 1) * 5, jj * HP:(jj + 1) * HP].set(wp.T)
    b128 = jnp.tile(jnp.pad(pos_b, (0, HP - H)), 8).reshape(1, 128)

    table_pk = _project_table(pre40, w2, b128)
    table16 = table_pk.reshape(T_PAD, HP)
    q2d, kv2d = _project_qkv(x, q_w, q_b, kv_w, kv_b)

    bk1 = blank_k.reshape(1, DIM)
    bv1 = blank_v.reshape(1, DIM)
    pb1 = proj_b.reshape(1, DIM)
    outs = []
    tok0 = jnp.zeros((128,), jnp.float32)
    for q in range(NSPLIT):
        tok = tok0
        if q >= 2:
            tok, _ = lax.optimization_barrier((tok0, outs[q - 2]))
        bias_q = _get_sc_gather(q)(table16, idxnn, tok)
        outs.append(_attention(q, q2d, kv2d, bias_q, bk1, bv1, proj_w, pb1))
    out = jnp.concatenate(outs, axis=0)
    return out.reshape(1, N, DIM)


# speedup vs baseline: 1.2662x; 1.0156x over previous
import functools

import jax
import jax.numpy as jnp
from jax import lax
from jax.experimental import pallas as pl
from jax.experimental.pallas import tpu as pltpu
from jax.experimental.pallas import tpu_sc as plsc

N = 2048
DIM = 768
H = 12
CH = 64
HP = 16
T_PAD = 1 << 20

NC = 2
NS = 16
NW = NC * NS
PER_W = (N * N) // NW
CHUNK = 2048
N_CHUNKS = PER_W // CHUNK
ROWS128 = CHUNK // 128
PSTRIDE = CHUNK + 8

BQ = 128
BK = 512
SCALE = CH ** -0.5
NEG_BIG = -1e30


def _table_body(pt_ref, w_ref, b_ref, o_ref):
    o_ref[...] = (
        lax.dot_general(pt_ref[...], w_ref[...], (((1,), (0,)), ((), ())),
                        preferred_element_type=jnp.float32)
        + b_ref[...]
    )


def _project_table(pre40, w2, b128):
    bt = 1024
    return pl.pallas_call(
        _table_body,
        grid=(T_PAD // 8 // bt,),
        in_specs=[
            pl.BlockSpec((bt, 40), lambda i: (i, 0)),
            pl.BlockSpec((40, 128), lambda i: (0, 0)),
            pl.BlockSpec((1, 128), lambda i: (0, 0)),
        ],
        out_specs=pl.BlockSpec((bt, 128), lambda i: (i, 0)),
        out_shape=jax.ShapeDtypeStruct((T_PAD // 8, 128), jnp.float32),
    )(pre40, w2, b128)


def _qkv_body(x_ref, qw_ref, qb_ref, kvw_ref, kvb_ref, q_ref, kv_ref):
    x = x_ref[...]
    q = lax.dot_general(x, qw_ref[...], (((1,), (1,)), ((), ())),
                        preferred_element_type=jnp.float32) + qb_ref[...]
    q_ref[...] = q * SCALE
    kv_ref[...] = lax.dot_general(x, kvw_ref[...], (((1,), (1,)), ((), ())),
                                  preferred_element_type=jnp.float32) + kvb_ref[...]


def _project_qkv(x, q_w, q_b, kv_w, kv_b):
    bn = 256
    return pl.pallas_call(
        _qkv_body,
        grid=(N // bn,),
        in_specs=[
            pl.BlockSpec((bn, DIM), lambda i: (i, 0)),
            pl.BlockSpec((DIM, DIM), lambda i: (0, 0)),
            pl.BlockSpec((1, DIM), lambda i: (0, 0)),
            pl.BlockSpec((2 * DIM, DIM), lambda i: (0, 0)),
            pl.BlockSpec((1, 2 * DIM), lambda i: (0, 0)),
        ],
        out_specs=[
            pl.BlockSpec((bn, DIM), lambda i: (i, 0)),
            pl.BlockSpec((bn, 2 * DIM), lambda i: (i, 0)),
        ],
        out_shape=[
            jax.ShapeDtypeStruct((N, DIM), jnp.float32),
            jax.ShapeDtypeStruct((N, 2 * DIM), jnp.float32),
        ],
    )(x, q_w, q_b.reshape(1, DIM), kv_w, kv_b.reshape(1, 2 * DIM))


NSPLIT = 8


def _sc_gather_body(part, table_hbm, idx_hbm, tok_hbm, out_hbm, idx_v, rows_v,
                    planes_v, sem0, sem1):
    del tok_hbm
    wid = lax.axis_index("s") * NC + lax.axis_index("c")
    nch = N_CHUNKS // NSPLIT
    row0 = part * (N // NSPLIT) + wid * nch
    sems = (sem0, sem1)

    def stage(slot, row):
        pltpu.sync_copy(idx_hbm.at[row], idx_v.at[slot])
        for r in range(ROWS128):
            pltpu.async_copy(
                table_hbm.at[idx_v.at[slot, pl.ds(r * 128, 128)]],
                rows_v.at[slot, pl.ds(r * 128, 128)], sems[slot])

    def drain(slot):
        pltpu.make_async_copy(table_hbm.at[pl.ds(0, CHUNK)],
                              rows_v.at[slot], sems[slot]).wait()

    def process(slot, row):
        strides = lax.iota(jnp.int32, 16) * PSTRIDE

        def g_body(g, carry2):
            p0 = g * 8
            for u in range(8):
                vals = rows_v[slot, p0 + u, :]
                plsc.store_scatter(planes_v, [strides + (p0 + u)], vals)
            return carry2

        lax.fori_loop(0, CHUNK // 8, g_body, 0, unroll=False)
        for h in range(H):
            pltpu.sync_copy(planes_v.at[pl.ds(h * PSTRIDE, CHUNK)],
                            out_hbm.at[h, row - part * (N // NSPLIT)])

    stage(0, row0)

    def pair_body(c2, carry):
        base_row = row0 + c2 * 2
        stage(1, base_row + 1)
        drain(0)
        process(0, base_row)

        @pl.when(c2 * 2 + 2 < nch)
        def _prefetch():
            stage(0, base_row + 2)

        drain(1)
        process(1, base_row + 1)
        return carry

    lax.fori_loop(0, nch // 2, pair_body, 0, unroll=False)


@functools.cache
def _get_sc_gather(part):
    return pl.kernel(
        functools.partial(_sc_gather_body, part),
        out_type=jax.ShapeDtypeStruct((H, N // NSPLIT, N), jnp.float32),
        mesh=plsc.VectorSubcoreMesh(core_axis_name="c", subcore_axis_name="s"),
        scratch_types=[
            pltpu.VMEM((2, CHUNK), jnp.int32),
            pltpu.VMEM((2, CHUNK, HP), jnp.float32),
            pltpu.VMEM((HP * PSTRIDE,), jnp.float32),
            pltpu.SemaphoreType.DMA,
            pltpu.SemaphoreType.DMA,
        ],
        compiler_params=pltpu.CompilerParams(needs_layout_passes=False,
                                             use_tc_tiling_on_sc=False),
    )


def _attn_body(q_ref, kv_ref, bias_ref, bk_ref, bv_ref, pw_ref, pb_ref,
               o_ref, acc_ref, m_ref, l_ref):
    j = pl.program_id(1)

    @pl.when(j == 0)
    def _init():
        m_ref[...] = jnp.full_like(m_ref, NEG_BIG)
        l_ref[...] = jnp.zeros_like(l_ref)
        acc_ref[...] = jnp.zeros_like(acc_ref)

    q = q_ref[...]
    kv = kv_ref[pl.ds(j * BK, BK), :]
    for h in range(H):
        q_h = q[:, h * CH:(h + 1) * CH]
        k_h = kv[:, h * 2 * CH:h * 2 * CH + CH]
        v_h = kv[:, h * 2 * CH + CH:h * 2 * CH + 2 * CH]
        s = lax.dot_general(q_h, k_h, (((1,), (1,)), ((), ())),
                            preferred_element_type=jnp.float32)
        s = s + bias_ref[h]
        m_old = m_ref[:, h:h + 1]
        m_new = jnp.maximum(m_old, jnp.max(s, axis=1, keepdims=True))
        alpha = jnp.exp(m_old - m_new)
        p = jnp.exp(s - m_new)
        l_ref[:, h:h + 1] = (l_ref[:, h:h + 1] * alpha
                             + jnp.sum(p, axis=1, keepdims=True))
        acc_ref[:, h * CH:(h + 1) * CH] = (
            acc_ref[:, h * CH:(h + 1) * CH] * alpha
            + lax.dot_general(p, v_h, (((1,), (0,)), ((), ())),
                              preferred_element_type=jnp.float32))
        m_ref[:, h:h + 1] = m_new

    @pl.when(j == pl.num_programs(1) - 1)
    def _fin():
        cols = []
        for h in range(H):
            q_h = q[:, h * CH:(h + 1) * CH]
            bl = jnp.sum(q_h * bk_ref[:, h * CH:(h + 1) * CH],
                         axis=1, keepdims=True)
            m_old = m_ref[:, h:h + 1]
            m_fin = jnp.maximum(m_old, bl)
            a0 = jnp.exp(m_old - m_fin)
            ab = jnp.exp(bl - m_fin)
            l_fin = l_ref[:, h:h + 1] * a0 + ab
            acc_h = (acc_ref[:, h * CH:(h + 1) * CH] * a0
                     + ab * bv_ref[:, h * CH:(h + 1) * CH])
            cols.append(acc_h / l_fin)
        obuf = jnp.concatenate(cols, axis=1)
        o_ref[...] = (
            lax.dot_general(obuf, pw_ref[...], (((1,), (1,)), ((), ())),
                            preferred_element_type=jnp.float32)
            + pb_ref[...]
        )


def _attention(part, q2d, kv2d, bias3, blank_k, blank_v, proj_w, proj_b):
    ioff = part * (N // NSPLIT) // BQ
    return pl.pallas_call(
        _attn_body,
        grid=(N // NSPLIT // BQ, N // BK),
        in_specs=[
            pl.BlockSpec((BQ, DIM), lambda i, j: (i + ioff, 0)),
            pl.BlockSpec((N, 2 * DIM), lambda i, j: (0, 0)),
            pl.BlockSpec((H, BQ, BK), lambda i, j: (0, i, j)),
            pl.BlockSpec((1, DIM), lambda i, j: (0, 0)),
            pl.BlockSpec((1, DIM), lambda i, j: (0, 0)),
            pl.BlockSpec((DIM, DIM), lambda i, j: (0, 0)),
            pl.BlockSpec((1, DIM), lambda i, j: (0, 0)),
        ],
        out_specs=pl.BlockSpec((BQ, DIM), lambda i, j: (i, 0)),
        out_shape=jax.ShapeDtypeStruct((N // NSPLIT, DIM), jnp.float32),
        scratch_shapes=[
            pltpu.VMEM((BQ, DIM), jnp.float32),
            pltpu.VMEM((BQ, HP), jnp.float32),
            pltpu.VMEM((BQ, HP), jnp.float32),
        ],
        compiler_params=pltpu.CompilerParams(
            dimension_semantics=("arbitrary", "arbitrary")),
    )(q2d, kv2d, bias3, blank_k, blank_v, proj_w, proj_b)


def kernel(feat, member_idx, cluster_mask, pe_idx, global_attn, q_w, q_b,
           kv_w, kv_b, blank_k, blank_v, pos_w, pos_b, proj_w, proj_b,
           pre_table):
    x = feat[0]
    idxnn = pe_idx.reshape(N, N).astype(jnp.int32)

    t = pre_table.shape[0]
    flat = jnp.pad(pre_table.reshape(-1), (0, (T_PAD - t) * 5))
    pre40 = flat.reshape(T_PAD // 8, 40)
    wp = jnp.pad(pos_w, ((0, HP - H), (0, 0)))
    w2 = jnp.zeros((40, 128), jnp.float32)
    for jj in range(8):
        w2 = w2.at[jj * 5:(jj + 1) * 5, jj * HP:(jj + 1) * HP].set(wp.T)
    b128 = jnp.tile(jnp.pad(pos_b, (0, HP - H)), 8).reshape(1, 128)

    table_pk = _project_table(pre40, w2, b128)
    table16 = table_pk.reshape(T_PAD, HP)
    q2d, kv2d = _project_qkv(x, q_w, q_b, kv_w, kv_b)

    bk1 = blank_k.reshape(1, DIM)
    bv1 = blank_v.reshape(1, DIM)
    pb1 = proj_b.reshape(1, DIM)
    outs = []
    tok0 = jnp.zeros((128,), jnp.float32)
    for q in range(NSPLIT):
        tok = tok0
        if q >= 2:
            tok, _ = lax.optimization_barrier((tok0, outs[q - 2]))
        bias_q = _get_sc_gather(q)(table16, idxnn, tok)
        outs.append(_attention(q, q2d, kv2d, bias_q, bk1, bv1, proj_w, pb1))
    out = jnp.concatenate(outs, axis=0)
    return out.reshape(1, N, DIM)


# speedup vs baseline: 1.2957x; 1.0233x over previous
import functools

import jax
import jax.numpy as jnp
from jax import lax
from jax.experimental import pallas as pl
from jax.experimental.pallas import tpu as pltpu
from jax.experimental.pallas import tpu_sc as plsc

N = 2048
DIM = 768
H = 12
CH = 64
HP = 16
T_PAD = 1 << 20

NC = 2
NS = 16
NW = NC * NS
PER_W = (N * N) // NW
CHUNK = 2048
N_CHUNKS = PER_W // CHUNK
ROWS128 = CHUNK // 128
PSTRIDE = CHUNK + 8

BQ = 128
BK = 512
SCALE = CH ** -0.5
NEG_BIG = -1e30


def _table_body(pt_ref, w_ref, b_ref, o_ref):
    o_ref[...] = (
        lax.dot_general(pt_ref[...], w_ref[...], (((1,), (0,)), ((), ())),
                        preferred_element_type=jnp.float32)
        + b_ref[...]
    )


def _project_table(pre40, w2, b128):
    bt = 1024
    return pl.pallas_call(
        _table_body,
        grid=(T_PAD // 8 // bt,),
        in_specs=[
            pl.BlockSpec((bt, 40), lambda i: (i, 0)),
            pl.BlockSpec((40, 128), lambda i: (0, 0)),
            pl.BlockSpec((1, 128), lambda i: (0, 0)),
        ],
        out_specs=pl.BlockSpec((bt, 128), lambda i: (i, 0)),
        out_shape=jax.ShapeDtypeStruct((T_PAD // 8, 128), jnp.float32),
    )(pre40, w2, b128)


def _qkv_body(x_ref, qw_ref, qb_ref, kvw_ref, kvb_ref, q_ref, kv_ref):
    x = x_ref[...]
    q = lax.dot_general(x, qw_ref[...], (((1,), (1,)), ((), ())),
                        preferred_element_type=jnp.float32) + qb_ref[...]
    q_ref[...] = q * SCALE
    kv_ref[...] = lax.dot_general(x, kvw_ref[...], (((1,), (1,)), ((), ())),
                                  preferred_element_type=jnp.float32) + kvb_ref[...]


def _project_qkv(x, q_w, q_b, kv_w, kv_b):
    bn = 256
    return pl.pallas_call(
        _qkv_body,
        grid=(N // bn,),
        in_specs=[
            pl.BlockSpec((bn, DIM), lambda i: (i, 0)),
            pl.BlockSpec((DIM, DIM), lambda i: (0, 0)),
            pl.BlockSpec((1, DIM), lambda i: (0, 0)),
            pl.BlockSpec((2 * DIM, DIM), lambda i: (0, 0)),
            pl.BlockSpec((1, 2 * DIM), lambda i: (0, 0)),
        ],
        out_specs=[
            pl.BlockSpec((bn, DIM), lambda i: (i, 0)),
            pl.BlockSpec((bn, 2 * DIM), lambda i: (i, 0)),
        ],
        out_shape=[
            jax.ShapeDtypeStruct((N, DIM), jnp.float32),
            jax.ShapeDtypeStruct((N, 2 * DIM), jnp.float32),
        ],
    )(x, q_w, q_b.reshape(1, DIM), kv_w, kv_b.reshape(1, 2 * DIM))


NSPLIT = 8


PLANES_L = H * PSTRIDE + 64


def _sc_gather_body(part, table_hbm, idx_hbm, tok_hbm, out_hbm, idx_v, rows_v,
                    planes_v, sem0, sem1, semo0, semo1):
    del tok_hbm
    wid = lax.axis_index("s") * NC + lax.axis_index("c")
    nch = N_CHUNKS // NSPLIT
    row0 = part * (N // NSPLIT) + wid * nch
    sems = (sem0, sem1)
    semos = (semo0, semo1)

    def stage(slot, row):
        pltpu.sync_copy(idx_hbm.at[row], idx_v.at[slot])
        for r in range(ROWS128):
            pltpu.async_copy(
                table_hbm.at[idx_v.at[slot, pl.ds(r * 128, 128)]],
                rows_v.at[slot, pl.ds(r * 128, 128)], sems[slot])

    def drain(slot):
        pltpu.make_async_copy(table_hbm.at[pl.ds(0, CHUNK)],
                              rows_v.at[slot], sems[slot]).wait()

    def drain_out(slot):
        for h in range(H):
            pltpu.make_async_copy(
                out_hbm.at[h, 0],
                planes_v.at[slot, pl.ds(h * PSTRIDE, CHUNK)],
                semos[slot]).wait()

    lanes = lax.iota(jnp.int32, 16)
    strides = jnp.where(lanes < H, lanes * PSTRIDE,
                        H * PSTRIDE + (lanes - H) * 8)

    def process(slot, row, c2):
        @pl.when(c2 > 0)
        def _():
            drain_out(slot)

        def g_body(g, carry2):
            p0 = g * 8
            for u in range(8):
                vals = rows_v[slot, p0 + u, :]
                plsc.store_scatter(planes_v.at[slot],
                                   [strides + (p0 + u)], vals)
            return carry2

        lax.fori_loop(0, CHUNK // 8, g_body, 0, unroll=False)
        for h in range(H):
            pltpu.async_copy(planes_v.at[slot, pl.ds(h * PSTRIDE, CHUNK)],
                             out_hbm.at[h, row - part * (N // NSPLIT)],
                             semos[slot])

    stage(0, row0)

    def pair_body(c2, carry):
        base_row = row0 + c2 * 2
        stage(1, base_row + 1)
        drain(0)
        process(0, base_row, c2)

        @pl.when(c2 * 2 + 2 < nch)
        def _prefetch():
            stage(0, base_row + 2)

        drain(1)
        process(1, base_row + 1, c2)
        return carry

    lax.fori_loop(0, nch // 2, pair_body, 0, unroll=False)
    drain_out(0)
    drain_out(1)


@functools.cache
def _get_sc_gather(part):
    return pl.kernel(
        functools.partial(_sc_gather_body, part),
        out_type=jax.ShapeDtypeStruct((H, N // NSPLIT, N), jnp.float32),
        mesh=plsc.VectorSubcoreMesh(core_axis_name="c", subcore_axis_name="s"),
        scratch_types=[
            pltpu.VMEM((2, CHUNK), jnp.int32),
            pltpu.VMEM((2, CHUNK, HP), jnp.float32),
            pltpu.VMEM((2, PLANES_L), jnp.float32),
            pltpu.SemaphoreType.DMA,
            pltpu.SemaphoreType.DMA,
            pltpu.SemaphoreType.DMA,
            pltpu.SemaphoreType.DMA,
        ],
        compiler_params=pltpu.CompilerParams(needs_layout_passes=False,
                                             use_tc_tiling_on_sc=False),
    )


def _attn_body(q_ref, kv_ref, bias_ref, bk_ref, bv_ref, pw_ref, pb_ref,
               o_ref, acc_ref, m_ref, l_ref):
    j = pl.program_id(1)

    @pl.when(j == 0)
    def _init():
        m_ref[...] = jnp.full_like(m_ref, NEG_BIG)
        l_ref[...] = jnp.zeros_like(l_ref)
        acc_ref[...] = jnp.zeros_like(acc_ref)

    q = q_ref[...]
    kv = kv_ref[pl.ds(j * BK, BK), :]
    for h in range(H):
        q_h = q[:, h * CH:(h + 1) * CH]
        k_h = kv[:, h * 2 * CH:h * 2 * CH + CH]
        v_h = kv[:, h * 2 * CH + CH:h * 2 * CH + 2 * CH]
        s = lax.dot_general(q_h, k_h, (((1,), (1,)), ((), ())),
                            preferred_element_type=jnp.float32)
        s = s + bias_ref[h]
        m_old = m_ref[:, h:h + 1]
        m_new = jnp.maximum(m_old, jnp.max(s, axis=1, keepdims=True))
        alpha = jnp.exp(m_old - m_new)
        p = jnp.exp(s - m_new)
        l_ref[:, h:h + 1] = (l_ref[:, h:h + 1] * alpha
                             + jnp.sum(p, axis=1, keepdims=True))
        acc_ref[:, h * CH:(h + 1) * CH] = (
            acc_ref[:, h * CH:(h + 1) * CH] * alpha
            + lax.dot_general(p, v_h, (((1,), (0,)), ((), ())),
                              preferred_element_type=jnp.float32))
        m_ref[:, h:h + 1] = m_new

    @pl.when(j == pl.num_programs(1) - 1)
    def _fin():
        cols = []
        for h in range(H):
            q_h = q[:, h * CH:(h + 1) * CH]
            bl = jnp.sum(q_h * bk_ref[:, h * CH:(h + 1) * CH],
                         axis=1, keepdims=True)
            m_old = m_ref[:, h:h + 1]
            m_fin = jnp.maximum(m_old, bl)
            a0 = jnp.exp(m_old - m_fin)
            ab = jnp.exp(bl - m_fin)
            l_fin = l_ref[:, h:h + 1] * a0 + ab
            acc_h = (acc_ref[:, h * CH:(h + 1) * CH] * a0
                     + ab * bv_ref[:, h * CH:(h + 1) * CH])
            cols.append(acc_h / l_fin)
        obuf = jnp.concatenate(cols, axis=1)
        o_ref[...] = (
            lax.dot_general(obuf, pw_ref[...], (((1,), (1,)), ((), ())),
                            preferred_element_type=jnp.float32)
            + pb_ref[...]
        )


def _attention(part, q2d, kv2d, bias3, blank_k, blank_v, proj_w, proj_b):
    ioff = part * (N // NSPLIT) // BQ
    return pl.pallas_call(
        _attn_body,
        grid=(N // NSPLIT // BQ, N // BK),
        in_specs=[
            pl.BlockSpec((BQ, DIM), lambda i, j: (i + ioff, 0)),
            pl.BlockSpec((N, 2 * DIM), lambda i, j: (0, 0)),
            pl.BlockSpec((H, BQ, BK), lambda i, j: (0, i, j)),
            pl.BlockSpec((1, DIM), lambda i, j: (0, 0)),
            pl.BlockSpec((1, DIM), lambda i, j: (0, 0)),
            pl.BlockSpec((DIM, DIM), lambda i, j: (0, 0)),
            pl.BlockSpec((1, DIM), lambda i, j: (0, 0)),
        ],
        out_specs=pl.BlockSpec((BQ, DIM), lambda i, j: (i, 0)),
        out_shape=jax.ShapeDtypeStruct((N // NSPLIT, DIM), jnp.float32),
        scratch_shapes=[
            pltpu.VMEM((BQ, DIM), jnp.float32),
            pltpu.VMEM((BQ, HP), jnp.float32),
            pltpu.VMEM((BQ, HP), jnp.float32),
        ],
        compiler_params=pltpu.CompilerParams(
            dimension_semantics=("arbitrary", "arbitrary")),
    )(q2d, kv2d, bias3, blank_k, blank_v, proj_w, proj_b)


def kernel(feat, member_idx, cluster_mask, pe_idx, global_attn, q_w, q_b,
           kv_w, kv_b, blank_k, blank_v, pos_w, pos_b, proj_w, proj_b,
           pre_table):
    x = feat[0]
    idxnn = pe_idx.reshape(N, N).astype(jnp.int32)

    t = pre_table.shape[0]
    flat = jnp.pad(pre_table.reshape(-1), (0, (T_PAD - t) * 5))
    pre40 = flat.reshape(T_PAD // 8, 40)
    wp = jnp.pad(pos_w, ((0, HP - H), (0, 0)))
    w2 = jnp.zeros((40, 128), jnp.float32)
    for jj in range(8):
        w2 = w2.at[jj * 5:(jj + 1) * 5, jj * HP:(jj + 1) * HP].set(wp.T)
    b128 = jnp.tile(jnp.pad(pos_b, (0, HP - H)), 8).reshape(1, 128)

    table_pk = _project_table(pre40, w2, b128)
    table16 = table_pk.reshape(T_PAD, HP)
    q2d, kv2d = _project_qkv(x, q_w, q_b, kv_w, kv_b)

    bk1 = blank_k.reshape(1, DIM)
    bv1 = blank_v.reshape(1, DIM)
    pb1 = proj_b.reshape(1, DIM)
    outs = []
    tok0 = jnp.zeros((128,), jnp.float32)
    for q in range(NSPLIT):
        tok = tok0
        if q >= 2:
            tok, _ = lax.optimization_barrier((tok0, outs[q - 2]))
        bias_q = _get_sc_gather(q)(table16, idxnn, tok)
        outs.append(_attention(q, q2d, kv2d, bias_q, bk1, bv1, proj_w, pb1))
    out = jnp.concatenate(outs, axis=0)
    return out.reshape(1, N, DIM)
